# baseline (speedup 1.0000x reference)
import functools

import jax
import jax.numpy as jnp
from jax import lax
from jax.experimental import pallas as pl
from jax.experimental.pallas import tpu as pltpu
from jax.experimental.pallas import tpu_sc as plsc

B = 16384
NCOL = 28
DIM = 64
NNUM = NCOL - 2
FLAT = NCOL * DIM

NC = 2
NS = 16
NW = NC * NS
BPW = B // NW
IDX_CHUNK = 128
NCHUNK = BPW // IDX_CHUNK

TC_BLK = 512


def _sc_gather_body(stock_hbm, time_hbm, sidx_hbm, tidx_hbm,
                    es_hbm, et_hbm,
                    sidx_v, tidx_v, srows_v, trows_v, sem):
    wid = lax.axis_index("s") * NC + lax.axis_index("c")
    row0 = wid * NCHUNK
    base = wid * BPW

    pltpu.sync_copy(sidx_hbm.at[pl.ds(row0, NCHUNK)], sidx_v)
    pltpu.sync_copy(tidx_hbm.at[pl.ds(row0, NCHUNK)], tidx_v)

    copies = []
    for c in range(NCHUNK):
        copies.append(pltpu.async_copy(
            stock_hbm.at[sidx_v.at[c]],
            srows_v.at[pl.ds(c * IDX_CHUNK, IDX_CHUNK)], sem))
        copies.append(pltpu.async_copy(
            time_hbm.at[tidx_v.at[c]],
            trows_v.at[pl.ds(c * IDX_CHUNK, IDX_CHUNK)], sem))
    for cp in copies:
        cp.wait()

    pltpu.sync_copy(srows_v, es_hbm.at[pl.ds(base, BPW)])
    pltpu.sync_copy(trows_v, et_hbm.at[pl.ds(base, BPW)])


@functools.cache
def _sc_gather():
    return pl.kernel(
        _sc_gather_body,
        out_type=(jax.ShapeDtypeStruct((B, DIM), jnp.float32),
                  jax.ShapeDtypeStruct((B, DIM), jnp.float32)),
        mesh=plsc.VectorSubcoreMesh(core_axis_name="c", subcore_axis_name="s",
                                    num_cores=NC, num_subcores=NS),
        scratch_types=(
            pltpu.VMEM((NCHUNK, IDX_CHUNK), jnp.int32),
            pltpu.VMEM((NCHUNK, IDX_CHUNK), jnp.int32),
            pltpu.VMEM((BPW, DIM), jnp.float32),
            pltpu.VMEM((BPW, DIM), jnp.float32),
            pltpu.SemaphoreType.DMA,
        ),
        compiler_params=pltpu.CompilerParams(use_tc_tiling_on_sc=False),
    )


def _tc_body(x_ref, es_ref, et_ref, w2_ref, b2_ref, out_ref):
    out_ref[:, 0:DIM] = es_ref[...]
    out_ref[:, DIM:2 * DIM] = et_ref[...]
    lane = lax.broadcasted_iota(jnp.int32, (1, 2 * DIM), 1)
    left = lane < DIM
    w2 = w2_ref[...]
    b2 = b2_ref[...]
    for m in range(NNUM // 2):
        a = x_ref[:, 2 + 2 * m:3 + 2 * m]
        c = x_ref[:, 3 + 2 * m:4 + 2 * m]
        pair = jnp.where(left, a, c)
        out_ref[:, (m + 1) * 128:(m + 2) * 128] = jnp.maximum(
            pair * w2 + b2, 0.0)


def _tc_assemble(x, es, et, w2, b2):
    grid = B // TC_BLK
    return pl.pallas_call(
        _tc_body,
        grid=(grid,),
        in_specs=[
            pl.BlockSpec((TC_BLK, NCOL), lambda i: (i, 0)),
            pl.BlockSpec((TC_BLK, DIM), lambda i: (i, 0)),
            pl.BlockSpec((TC_BLK, DIM), lambda i: (i, 0)),
            pl.BlockSpec((1, 2 * DIM), lambda i: (0, 0)),
            pl.BlockSpec((1, 2 * DIM), lambda i: (0, 0)),
        ],
        out_specs=pl.BlockSpec((TC_BLK, FLAT), lambda i: (i, 0)),
        out_shape=jax.ShapeDtypeStruct((B, FLAT), jnp.float32),
    )(x, es, et, w2, b2)


def kernel(x, stock_table, time_table, W, b):
    s_idx = x[:, 0].astype(jnp.int32).reshape(B // 128, 128)
    t_idx = x[:, 1].astype(jnp.int32).reshape(B // 128, 128)
    w2 = jnp.concatenate([W, W], axis=1)
    b2 = jnp.concatenate([b, b]).reshape(1, 2 * DIM)

    es, et = _sc_gather()(stock_table, time_table, s_idx, t_idx)
    flat = _tc_assemble(x, es, et, w2, b2)
    return flat.reshape(B, NCOL, DIM)
